# baseline (speedup 1.0000x reference)
import functools

import jax
import jax.numpy as jnp
from jax import lax
from jax.experimental import pallas as pl
from jax.experimental.pallas import tpu as pltpu
from jax.experimental.pallas import tpu_sc as plsc

N = 10000
E = 320000
IN_DIM = 128
HID1 = 32
HID2 = 16

NC = 2
NS = 16
LANES = 16
NW = NC * NS

CHUNK = 128
NCHG = E // CHUNK
NCH_LO = NCHG // NW
NCH_REM = NCHG - NCH_LO * NW
NPIPE = 9
_PIPE = 8

RPT = 632
NROWS = RPT * NS

_mesh = plsc.VectorSubcoreMesh(
    core_axis_name="c", subcore_axis_name="s", num_cores=NC, num_subcores=NS
)

_sc_params = pltpu.CompilerParams(use_tc_tiling_on_sc=False,
                                  disable_bounds_checks=True)


def _slab_range(wid):
    base = wid * NCH_LO + jnp.minimum(wid, NCH_REM)
    nch = NCH_LO + jnp.where(wid < NCH_REM, 1, 0)
    return base, nch


HLN = 8


@functools.partial(
    pl.kernel,
    out_type=jax.ShapeDtypeStruct((NC, NROWS, HLN), jnp.float32),
    mesh=_mesh,
    scratch_types=[
        pltpu.VMEM((NCH_LO + 1, 2, CHUNK), jnp.int32),
        pltpu.VMEM((CHUNK, HLN), jnp.float32),
        pltpu.VMEM_SHARED((NROWS, HLN), jnp.float32),
    ],
    compiler_params=_sc_params,
)
def _deg_kernel(ei_hbm, ones_hbm, zeros_hbm, hist_hbm, ei_v, ones_v, acc):
    c = lax.axis_index("c")
    s = lax.axis_index("s")
    wid = c * NS + s
    base, nch = _slab_range(wid)
    pltpu.sync_copy(ei_hbm.at[pl.ds(base, NCH_LO)], ei_v.at[pl.ds(0, NCH_LO)])

    @pl.when(wid < NCH_REM)
    def _():
        pltpu.sync_copy(ei_hbm.at[pl.ds(base + NCH_LO, 1)],
                        ei_v.at[pl.ds(NCH_LO, 1)])

    pltpu.sync_copy(ones_hbm, ones_v)
    pltpu.sync_copy(zeros_hbm, acc.at[pl.ds(s * RPT, RPT)])
    plsc.subcore_barrier()

    @pl.loop(0, nch)
    def _(j):
        pltpu.sync_copy(ones_v, acc.at[ei_v.at[j, 1]], add=True)

    plsc.subcore_barrier()
    pltpu.sync_copy(acc.at[pl.ds(s * RPT, RPT)], hist_hbm.at[c, pl.ds(s * RPT, RPT)])


def _make_scatter(D):
    @functools.partial(
        pl.kernel,
        out_type=jax.ShapeDtypeStruct((NC, NROWS, D), jnp.float32),
        mesh=_mesh,
        scratch_types=[
            pltpu.VMEM((NCH_LO + 1, 2, CHUNK), jnp.int32),
            pltpu.VMEM((_PIPE, CHUNK, D), jnp.float32),
            pltpu.VMEM_SHARED((NROWS, D), jnp.float32),
            pltpu.VMEM_SHARED((N, D), jnp.float32),
            pltpu.SemaphoreType.DMA((_PIPE,)),
            pltpu.SemaphoreType.DMA((_PIPE,)),
        ],
        compiler_params=_sc_params,
    )
    def scatter_kernel(g_hbm, ei_hbm, zeros_hbm, tmp_hbm,
                       ei_v, rows_v, acc, g_spm, gsem, ssem):
        c = lax.axis_index("c")
        s = lax.axis_index("s")
        wid = c * NS + s
        base, nch = _slab_range(wid)
        pltpu.sync_copy(ei_hbm.at[pl.ds(base, NCH_LO)],
                        ei_v.at[pl.ds(0, NCH_LO)])

        @pl.when(wid < NCH_REM)
        def _():
            pltpu.sync_copy(ei_hbm.at[pl.ds(base + NCH_LO, 1)],
                            ei_v.at[pl.ds(NCH_LO, 1)])

        pltpu.sync_copy(zeros_hbm, acc.at[pl.ds(s * RPT, RPT)])
        pltpu.sync_copy(g_hbm.at[pl.ds(s * (N // NS), N // NS)],
                        g_spm.at[pl.ds(s * (N // NS), N // NS)])
        plsc.subcore_barrier()

        @pl.loop(0, NPIPE)
        def _(jj):
            bb = jj * _PIPE
            for b in range(_PIPE):
                @pl.when(jj > 0)
                def _():
                    pltpu.make_async_copy(
                        rows_v.at[b], acc.at[ei_v.at[bb - _PIPE + b, 1]],
                        ssem.at[b]).wait()

                pltpu.async_copy(g_spm.at[ei_v.at[bb + b, 0]], rows_v.at[b],
                                 gsem.at[b])
            for b in range(_PIPE):
                pltpu.make_async_copy(g_spm.at[ei_v.at[bb + b, 0]],
                                      rows_v.at[b], gsem.at[b]).wait()
                pltpu.async_copy(rows_v.at[b], acc.at[ei_v.at[bb + b, 1]],
                                 ssem.at[b], add=True)

        for b in range(_PIPE):
            pltpu.make_async_copy(
                rows_v.at[b], acc.at[ei_v.at[(NPIPE - 1) * _PIPE + b, 1]],
                ssem.at[b]).wait()

        @pl.loop(NPIPE * _PIPE, nch)
        def _(j):
            pltpu.async_copy(g_spm.at[ei_v.at[j, 0]], rows_v.at[0],
                             gsem.at[0]).wait()
            pltpu.sync_copy(rows_v.at[0], acc.at[ei_v.at[j, 1]], add=True)

        plsc.subcore_barrier()
        pltpu.sync_copy(acc.at[pl.ds(s * RPT, RPT)],
                        tmp_hbm.at[c, pl.ds(s * RPT, RPT)])

    return scatter_kernel


_scatter_h1 = _make_scatter(HID1)
_scatter_h2 = _make_scatter(HID2)


def _tcb_body(x_ref, w_ref, hist_ref, g_ref, dis_ref):
    deg = (hist_ref[0, :N, :].sum(axis=1, keepdims=True)
           + hist_ref[1, :N, :].sum(axis=1, keepdims=True) + 1.0)
    dis = lax.rsqrt(deg)
    h = jnp.dot(x_ref[...], w_ref[...], preferred_element_type=jnp.float32)
    g_ref[...] = h * dis
    dis_ref[...] = dis


_tc_first = pl.pallas_call(
    _tcb_body,
    out_shape=(
        jax.ShapeDtypeStruct((N, HID1), jnp.float32),
        jax.ShapeDtypeStruct((N, 1), jnp.float32),
    ),
)


def _tcd_body(t_ref, g1_ref, dis_ref, b_ref, w_ref, g2_ref):
    h = jnp.maximum(
        (t_ref[0, :N, :] + t_ref[1, :N, :] + g1_ref[...]) * dis_ref[...]
        + b_ref[...],
        0.0,
    )
    g2_ref[...] = jnp.dot(h, w_ref[...], preferred_element_type=jnp.float32) * dis_ref[...]


_tc_mid = pl.pallas_call(
    _tcd_body,
    out_shape=jax.ShapeDtypeStruct((N, HID2), jnp.float32),
)


def _tcf_body(t_ref, g2_ref, dis_ref, b_ref, out_ref):
    out_ref[...] = ((t_ref[0, :N, :] + t_ref[1, :N, :] + g2_ref[...])
                    * dis_ref[...] + b_ref[...])


_tc_last = pl.pallas_call(
    _tcf_body,
    out_shape=jax.ShapeDtypeStruct((N, HID2), jnp.float32),
)


def kernel(x, edge_index, W1, b1, W2, b2):
    ei3 = jnp.swapaxes(edge_index.reshape(2, NCHG, CHUNK), 0, 1)

    ones = jnp.zeros((CHUNK, HLN), jnp.float32).at[:, 0].set(1.0)
    z16 = jnp.zeros((RPT, HLN), jnp.float32)
    z32 = jnp.zeros((RPT, HID1), jnp.float32)
    zh2 = jnp.zeros((RPT, HID2), jnp.float32)

    hist = _deg_kernel(ei3, ones, z16)
    g1, dis = _tc_first(x, W1, hist)
    t1 = _scatter_h1(g1, ei3, z32)
    g2 = _tc_mid(t1, g1, dis, b1.reshape(1, HID1), W2)
    t2 = _scatter_h2(g2, ei3, zh2)
    out = _tc_last(t2, g2, dis, b2.reshape(1, HID2))
    return out


# speedup vs baseline: 1.0516x; 1.0516x over previous
import functools

import jax
import jax.numpy as jnp
from jax import lax
from jax.experimental import pallas as pl
from jax.experimental.pallas import tpu as pltpu
from jax.experimental.pallas import tpu_sc as plsc

N = 10000
E = 320000
IN_DIM = 128
HID1 = 32
HID2 = 16

NC = 2
NS = 16
LANES = 16
NW = NC * NS

CHUNK = 128
NCHG = E // CHUNK
NCH_LO = NCHG // NW
NCH_REM = NCHG - NCH_LO * NW
NPIPE = 9
_PIPE = 8

RPT = 632
NROWS = RPT * NS

_mesh = plsc.VectorSubcoreMesh(
    core_axis_name="c", subcore_axis_name="s", num_cores=NC, num_subcores=NS
)

_sc_params = pltpu.CompilerParams(use_tc_tiling_on_sc=False,
                                  disable_bounds_checks=True)


def _slab_range(wid):
    base = wid * NCH_LO + jnp.minimum(wid, NCH_REM)
    nch = NCH_LO + jnp.where(wid < NCH_REM, 1, 0)
    return base, nch


HLN = 8


@functools.partial(
    pl.kernel,
    out_type=jax.ShapeDtypeStruct((NC, NROWS, HLN), jnp.float32),
    mesh=_mesh,
    scratch_types=[
        pltpu.VMEM((NCH_LO + 1, 2, CHUNK), jnp.int32),
        pltpu.VMEM((CHUNK, HLN), jnp.float32),
        pltpu.VMEM_SHARED((NROWS, HLN), jnp.float32),
    ],
    compiler_params=_sc_params,
)
def _deg_kernel(ei_hbm, ones_hbm, zeros_hbm, hist_hbm, ei_v, ones_v, acc):
    c = lax.axis_index("c")
    s = lax.axis_index("s")
    wid = c * NS + s
    base, nch = _slab_range(wid)
    pltpu.sync_copy(ei_hbm.at[pl.ds(base, NCH_LO)], ei_v.at[pl.ds(0, NCH_LO)])

    @pl.when(wid < NCH_REM)
    def _():
        pltpu.sync_copy(ei_hbm.at[pl.ds(base + NCH_LO, 1)],
                        ei_v.at[pl.ds(NCH_LO, 1)])

    pltpu.sync_copy(ones_hbm, ones_v)
    pltpu.sync_copy(zeros_hbm, acc.at[pl.ds(s * RPT, RPT)])
    plsc.subcore_barrier()

    @pl.loop(0, nch)
    def _(j):
        pltpu.sync_copy(ones_v, acc.at[ei_v.at[j, 1]], add=True)

    plsc.subcore_barrier()
    pltpu.sync_copy(acc.at[pl.ds(s * RPT, RPT)], hist_hbm.at[c, pl.ds(s * RPT, RPT)])


def _make_scatter(D):
    @functools.partial(
        pl.kernel,
        out_type=jax.ShapeDtypeStruct((NC, NROWS, D), jnp.float32),
        mesh=_mesh,
        scratch_types=[
            pltpu.VMEM((NCH_LO + 1, 2, CHUNK), jnp.int32),
            pltpu.VMEM((_PIPE, CHUNK, D), jnp.float32),
            pltpu.VMEM_SHARED((NROWS, D), jnp.float32),
            pltpu.VMEM_SHARED((N, D), jnp.float32),
            pltpu.SemaphoreType.DMA((_PIPE,)),
            pltpu.SemaphoreType.DMA((_PIPE,)),
        ],
        compiler_params=_sc_params,
    )
    def scatter_kernel(g_hbm, ei_hbm, zeros_hbm, tmp_hbm,
                       ei_v, rows_v, acc, g_spm, gsem, ssem):
        c = lax.axis_index("c")
        s = lax.axis_index("s")
        wid = c * NS + s
        base, nch = _slab_range(wid)
        pltpu.sync_copy(ei_hbm.at[pl.ds(base, NCH_LO)],
                        ei_v.at[pl.ds(0, NCH_LO)])

        @pl.when(wid < NCH_REM)
        def _():
            pltpu.sync_copy(ei_hbm.at[pl.ds(base + NCH_LO, 1)],
                            ei_v.at[pl.ds(NCH_LO, 1)])

        pltpu.sync_copy(zeros_hbm, acc.at[pl.ds(s * RPT, RPT)])
        pltpu.sync_copy(g_hbm.at[pl.ds(s * (N // NS), N // NS)],
                        g_spm.at[pl.ds(s * (N // NS), N // NS)])
        plsc.subcore_barrier()

        @pl.loop(0, NPIPE)
        def _(jj):
            bb = jj * _PIPE
            for b in range(_PIPE):
                pltpu.async_copy(g_spm.at[ei_v.at[bb + b, 0]], rows_v.at[b],
                                 gsem.at[b])
            for b in range(_PIPE):
                pltpu.make_async_copy(g_spm.at[ei_v.at[bb + b, 0]],
                                      rows_v.at[b], gsem.at[b]).wait()
                pltpu.async_copy(rows_v.at[b], acc.at[ei_v.at[bb + b, 1]],
                                 ssem.at[b], add=True)
            for b in range(_PIPE):
                pltpu.make_async_copy(rows_v.at[b],
                                      acc.at[ei_v.at[bb + b, 1]],
                                      ssem.at[b]).wait()

        @pl.loop(NPIPE * _PIPE, nch)
        def _(j):
            pltpu.async_copy(g_spm.at[ei_v.at[j, 0]], rows_v.at[0],
                             gsem.at[0]).wait()
            pltpu.sync_copy(rows_v.at[0], acc.at[ei_v.at[j, 1]], add=True)

        plsc.subcore_barrier()
        pltpu.sync_copy(acc.at[pl.ds(s * RPT, RPT)],
                        tmp_hbm.at[c, pl.ds(s * RPT, RPT)])

    return scatter_kernel


_scatter_h1 = _make_scatter(HID1)
_scatter_h2 = _make_scatter(HID2)


def _tcb_body(x_ref, w_ref, hist_ref, g_ref, dis_ref):
    deg = (hist_ref[0, :N, :].sum(axis=1, keepdims=True)
           + hist_ref[1, :N, :].sum(axis=1, keepdims=True) + 1.0)
    dis = lax.rsqrt(deg)
    h = jnp.dot(x_ref[...], w_ref[...], preferred_element_type=jnp.float32)
    g_ref[...] = h * dis
    dis_ref[...] = dis


_tc_first = pl.pallas_call(
    _tcb_body,
    out_shape=(
        jax.ShapeDtypeStruct((N, HID1), jnp.float32),
        jax.ShapeDtypeStruct((N, 1), jnp.float32),
    ),
)


def _tcd_body(t_ref, g1_ref, dis_ref, b_ref, w_ref, g2_ref):
    h = jnp.maximum(
        (t_ref[0, :N, :] + t_ref[1, :N, :] + g1_ref[...]) * dis_ref[...]
        + b_ref[...],
        0.0,
    )
    g2_ref[...] = jnp.dot(h, w_ref[...], preferred_element_type=jnp.float32) * dis_ref[...]


_tc_mid = pl.pallas_call(
    _tcd_body,
    out_shape=jax.ShapeDtypeStruct((N, HID2), jnp.float32),
)


def _tcf_body(t_ref, g2_ref, dis_ref, b_ref, out_ref):
    out_ref[...] = ((t_ref[0, :N, :] + t_ref[1, :N, :] + g2_ref[...])
                    * dis_ref[...] + b_ref[...])


_tc_last = pl.pallas_call(
    _tcf_body,
    out_shape=jax.ShapeDtypeStruct((N, HID2), jnp.float32),
)


def kernel(x, edge_index, W1, b1, W2, b2):
    ei3 = jnp.swapaxes(edge_index.reshape(2, NCHG, CHUNK), 0, 1)

    ones = jnp.zeros((CHUNK, HLN), jnp.float32).at[:, 0].set(1.0)
    z16 = jnp.zeros((RPT, HLN), jnp.float32)
    z32 = jnp.zeros((RPT, HID1), jnp.float32)
    zh2 = jnp.zeros((RPT, HID2), jnp.float32)

    hist = _deg_kernel(ei3, ones, z16)
    g1, dis = _tc_first(x, W1, hist)
    t1 = _scatter_h1(g1, ei3, z32)
    g2 = _tc_mid(t1, g1, dis, b1.reshape(1, HID1), W2)
    t2 = _scatter_h2(g2, ei3, zh2)
    out = _tc_last(t2, g2, dis, b2.reshape(1, HID2))
    return out
